# Initial kernel scaffold; baseline (speedup 1.0000x reference)
import jax
import jax.numpy as jnp
from jax.experimental import pallas as pl


def kernel(x, emb, W1, b1, W2, b2):
    raise NotImplementedError("write your pallas kernel here")


# speedup vs baseline: 1.9711x; 1.9711x over previous
import functools

import jax
import jax.numpy as jnp
from jax import lax
from jax.experimental import pallas as pl
from jax.experimental.pallas import tpu as pltpu
from jax.experimental.pallas import tpu_sc as plsc

B = 16384
L = 50
LP = 64
LD = 56
D = 64
NW = 32
WPW = B // NW
RB = 16
NBLK = WPW // RB

_mesh = plsc.VectorSubcoreMesh(core_axis_name="c", subcore_axis_name="s")


@functools.partial(
    pl.kernel,
    out_type=jax.ShapeDtypeStruct((B, D), jnp.float32),
    mesh=_mesh,
    scratch_types=[
        pltpu.VMEM((2, RB, LP), jnp.int32),
        pltpu.VMEM((2, RB, LP), jnp.float32),
        pltpu.VMEM((2, RB, LD, D), jnp.float32),
        pltpu.VMEM((RB, D), jnp.float32),
        pltpu.SemaphoreType.DMA((2,)),
    ],
    compiler_params=pltpu.CompilerParams(use_tc_tiling_on_sc=False),
)
def _pool_sc(types_hbm, areas_hbm, emb_hbm, out_hbm,
             idx_v, ar_v, rows_v, out_v, sem):
    wid = lax.axis_index("s") * 2 + lax.axis_index("c")
    base = wid * WPW

    def load_block(slot, blk):
        b0 = base + blk * RB
        pltpu.sync_copy(types_hbm.at[pl.ds(b0, RB)], idx_v.at[slot])
        pltpu.sync_copy(areas_hbm.at[pl.ds(b0, RB)], ar_v.at[slot])

        def fire(r, carry):
            pltpu.async_copy(
                emb_hbm.at[idx_v.at[slot, r, pl.ds(0, LD)]],
                rows_v.at[slot, r],
                sem.at[slot],
            )
            return carry

        lax.fori_loop(0, RB, fire, 0)

    def drain_block(slot):
        def drain(r, carry):
            pltpu.make_async_copy(
                emb_hbm.at[idx_v.at[slot, r, pl.ds(0, LD)]],
                rows_v.at[slot, r],
                sem.at[slot],
            ).wait()
            return carry

        lax.fori_loop(0, RB, drain, 0)

    def process_block(slot):
        def prow(r, carry):
            av = [ar_v[slot, r, pl.ds(16 * g, 16)] for g in range(4)]
            aj = [av[j // 16][j % 16] for j in range(L)]
            acc = [jnp.zeros((16,), jnp.float32) for _ in range(4)]
            for j in range(L):
                for k in range(4):
                    acc[k] = acc[k] + rows_v[slot, r, j, pl.ds(16 * k, 16)] * aj[j]
            for k in range(4):
                out_v[r, pl.ds(16 * k, 16)] = acc[k]
            return carry

        lax.fori_loop(0, RB, prow, 0)

    load_block(0, 0)
    load_block(1, 1)

    def pb_loop(pb, carry):
        for par in range(2):
            blk = pb * 2 + par
            b0 = base + blk * RB
            drain_block(par)
            process_block(par)
            pltpu.sync_copy(out_v, out_hbm.at[pl.ds(b0, RB)])

            @pl.when(blk + 2 < NBLK)
            def _():
                load_block(par, blk + 2)
        return carry

    lax.fori_loop(0, NBLK // 2, pb_loop, 0)


_MLP_BB = 2048


def _mlp_body(p_ref, ar_ref, w1_ref, b1_ref, w2_ref, b2_ref, o_ref):
    asum = jnp.sum(ar_ref[...], axis=1, keepdims=True)
    p = p_ref[...] / (asum + 1e-8)
    h = jnp.dot(p, w1_ref[...], preferred_element_type=jnp.float32)
    h = jnp.maximum(h + b1_ref[...], 0.0)
    o_ref[...] = (
        jnp.dot(h, w2_ref[...], preferred_element_type=jnp.float32) + b2_ref[...]
    )


def _mlp(pooled, areas_p, W1, b1, W2, b2):
    return pl.pallas_call(
        _mlp_body,
        grid=(B // _MLP_BB,),
        in_specs=[
            pl.BlockSpec((_MLP_BB, D), lambda i: (i, 0)),
            pl.BlockSpec((_MLP_BB, LP), lambda i: (i, 0)),
            pl.BlockSpec((D, 32), lambda i: (0, 0)),
            pl.BlockSpec((1, 32), lambda i: (0, 0)),
            pl.BlockSpec((32, 3), lambda i: (0, 0)),
            pl.BlockSpec((1, 3), lambda i: (0, 0)),
        ],
        out_specs=pl.BlockSpec((_MLP_BB, 3), lambda i: (i, 0)),
        out_shape=jax.ShapeDtypeStruct((B, 3), jnp.float32),
    )(pooled, areas_p, W1, b1, W2, b2)


def kernel(x, emb, W1, b1, W2, b2):
    types = x[:, 0, :].astype(jnp.int32)
    areas = x[:, 1, :]
    types_p = jnp.pad(types, ((0, 0), (0, LP - L)))
    areas_p = jnp.pad(areas, ((0, 0), (0, LP - L)))
    pooled = _pool_sc(types_p, areas_p, emb)
    return _mlp(pooled, areas_p, W1, b1.reshape(1, 32), W2, b2.reshape(1, 3))


# speedup vs baseline: 1.9760x; 1.0025x over previous
import functools

import jax
import jax.numpy as jnp
from jax import lax
from jax.experimental import pallas as pl
from jax.experimental.pallas import tpu as pltpu
from jax.experimental.pallas import tpu_sc as plsc

B = 16384
L = 50
LP = 64
LD = 56
D = 64
NW = 32
WPW = B // NW
RB = 16
NBLK = WPW // RB

_mesh = plsc.VectorSubcoreMesh(core_axis_name="c", subcore_axis_name="s")


@functools.partial(
    pl.kernel,
    out_type=jax.ShapeDtypeStruct((B, D), jnp.float32),
    mesh=_mesh,
    scratch_types=[
        pltpu.VMEM((2, RB, LP), jnp.int32),
        pltpu.VMEM((2, RB, LP), jnp.float32),
        pltpu.VMEM((2, RB, LD, D), jnp.float32),
        pltpu.VMEM((RB, D), jnp.float32),
        pltpu.SemaphoreType.DMA((2,)),
    ],
    compiler_params=pltpu.CompilerParams(use_tc_tiling_on_sc=False),
)
def _pool_sc(types_hbm, areas_hbm, emb_hbm, out_hbm,
             idx_v, ar_v, rows_v, out_v, sem):
    wid = lax.axis_index("s") * 2 + lax.axis_index("c")
    base = wid * WPW

    def load_block(slot, blk):
        b0 = base + blk * RB
        pltpu.sync_copy(types_hbm.at[pl.ds(b0, RB)], idx_v.at[slot])
        pltpu.sync_copy(areas_hbm.at[pl.ds(b0, RB)], ar_v.at[slot])

        def fire(r, carry):
            pltpu.async_copy(
                emb_hbm.at[idx_v.at[slot, r, pl.ds(0, LD)]],
                rows_v.at[slot, r],
                sem.at[slot],
            )
            return carry

        lax.fori_loop(0, RB, fire, 0)

    def drain_block(slot):
        def drain(r, carry):
            pltpu.make_async_copy(
                emb_hbm.at[idx_v.at[slot, r, pl.ds(0, LD)]],
                rows_v.at[slot, r],
                sem.at[slot],
            ).wait()
            return carry

        lax.fori_loop(0, RB, drain, 0)

    def process_block(slot):
        def prow(r, carry):
            av = [ar_v[slot, r, pl.ds(16 * g, 16)] for g in range(4)]
            aj = [av[j // 16][j % 16] for j in range(L)]
            acc = [aj[0] * av[k] for k in range(4)]
            for k in range(4):
                out_v[r, pl.ds(16 * k, 16)] = acc[k]
            return carry

        lax.fori_loop(0, RB, prow, 0)

    load_block(0, 0)
    load_block(1, 1)

    def pb_loop(pb, carry):
        for par in range(2):
            blk = pb * 2 + par
            b0 = base + blk * RB
            drain_block(par)
            process_block(par)
            pltpu.sync_copy(out_v, out_hbm.at[pl.ds(b0, RB)])

            @pl.when(blk + 2 < NBLK)
            def _():
                load_block(par, blk + 2)
        return carry

    lax.fori_loop(0, NBLK // 2, pb_loop, 0)


_MLP_BB = 2048


def _mlp_body(p_ref, ar_ref, w1_ref, b1_ref, w2_ref, b2_ref, o_ref):
    asum = jnp.sum(ar_ref[...], axis=1, keepdims=True)
    p = p_ref[...] / (asum + 1e-8)
    h = jnp.dot(p, w1_ref[...], preferred_element_type=jnp.float32)
    h = jnp.maximum(h + b1_ref[...], 0.0)
    o_ref[...] = (
        jnp.dot(h, w2_ref[...], preferred_element_type=jnp.float32) + b2_ref[...]
    )


def _mlp(pooled, areas_p, W1, b1, W2, b2):
    return pl.pallas_call(
        _mlp_body,
        grid=(B // _MLP_BB,),
        in_specs=[
            pl.BlockSpec((_MLP_BB, D), lambda i: (i, 0)),
            pl.BlockSpec((_MLP_BB, LP), lambda i: (i, 0)),
            pl.BlockSpec((D, 32), lambda i: (0, 0)),
            pl.BlockSpec((1, 32), lambda i: (0, 0)),
            pl.BlockSpec((32, 3), lambda i: (0, 0)),
            pl.BlockSpec((1, 3), lambda i: (0, 0)),
        ],
        out_specs=pl.BlockSpec((_MLP_BB, 3), lambda i: (i, 0)),
        out_shape=jax.ShapeDtypeStruct((B, 3), jnp.float32),
    )(pooled, areas_p, W1, b1, W2, b2)


def kernel(x, emb, W1, b1, W2, b2):
    types = x[:, 0, :].astype(jnp.int32)
    areas = x[:, 1, :]
    types_p = jnp.pad(types, ((0, 0), (0, LP - L)))
    areas_p = jnp.pad(areas, ((0, 0), (0, LP - L)))
    pooled = _pool_sc(types_p, areas_p, emb)
    return _mlp(pooled, areas_p, W1, b1.reshape(1, 32), W2, b2.reshape(1, 3))


# speedup vs baseline: 14.9801x; 7.5809x over previous
import functools

import jax
import jax.numpy as jnp
from jax import lax
from jax.experimental import pallas as pl
from jax.experimental.pallas import tpu as pltpu
from jax.experimental.pallas import tpu_sc as plsc

B = 16384
L = 50
LP = 64
LD = 56
D = 64
NW = 32
WPW = B // NW
RB = 16
NBLK = WPW // RB

_mesh = plsc.VectorSubcoreMesh(core_axis_name="c", subcore_axis_name="s")


@functools.partial(
    pl.kernel,
    out_type=jax.ShapeDtypeStruct((B, D), jnp.float32),
    mesh=_mesh,
    scratch_types=[
        pltpu.VMEM((2, RB, LP), jnp.int32),
        pltpu.VMEM((2, RB, LP), jnp.float32),
        pltpu.VMEM((2, RB, LD, D), jnp.float32),
        pltpu.VMEM((RB, D), jnp.float32),
        pltpu.SemaphoreType.DMA((2,)),
    ],
    compiler_params=pltpu.CompilerParams(use_tc_tiling_on_sc=False),
)
def _pool_sc(types_hbm, areas_hbm, emb_hbm, out_hbm,
             idx_v, ar_v, rows_v, out_v, sem):
    wid = lax.axis_index("s") * 2 + lax.axis_index("c")
    base = wid * WPW

    def load_block(slot, blk):
        b0 = base + blk * RB
        pltpu.sync_copy(types_hbm.at[pl.ds(b0, RB)], idx_v.at[slot])
        pltpu.sync_copy(areas_hbm.at[pl.ds(b0, RB)], ar_v.at[slot])

        def fire(r, carry):
            pltpu.async_copy(
                emb_hbm.at[idx_v.at[slot, r, pl.ds(0, LD)]],
                rows_v.at[slot, r],
                sem.at[slot],
            )
            return carry

        lax.fori_loop(0, RB, fire, 0)

    def drain_block(slot):
        def drain(r, carry):
            pltpu.make_async_copy(
                emb_hbm.at[idx_v.at[slot, r, pl.ds(0, LD)]],
                rows_v.at[slot, r],
                sem.at[slot],
            ).wait()
            return carry

        lax.fori_loop(0, RB, drain, 0)

    def process_block(slot):
        def prow(r, carry):
            av = [ar_v[slot, r, pl.ds(16 * g, 16)] for g in range(4)]
            aj = [av[j // 16][j % 16] for j in range(L)]
            acc = [jnp.zeros((16,), jnp.float32) for _ in range(4)]
            for j in range(L):
                for k in range(4):
                    acc[k] = acc[k] + rows_v[slot, r, j, pl.ds(16 * k, 16)] * aj[j]
            for k in range(4):
                out_v[r, pl.ds(16 * k, 16)] = acc[k]
            return carry

        lax.fori_loop(0, RB, prow, 0)

    load_block(0, 0)
    load_block(1, 1)

    def pb_loop(pb, carry):
        for par in range(2):
            blk = pb * 2 + par
            b0 = base + blk * RB
            drain_block(par)
            process_block(par)
            pltpu.sync_copy(out_v, out_hbm.at[pl.ds(b0, RB)])

            @pl.when(blk + 2 < NBLK)
            def _():
                load_block(par, blk + 2)
        return carry

    lax.fori_loop(0, NBLK // 2, pb_loop, 0)


_MLP_BB = 2048


def _mlp_body(p_ref, ar_ref, w1_ref, b1_ref, w2_ref, b2_ref, o_ref):
    asum = jnp.sum(ar_ref[...], axis=1, keepdims=True)
    p = p_ref[...] / (asum + 1e-8)
    h = jnp.dot(p, w1_ref[...], preferred_element_type=jnp.float32)
    h = jnp.maximum(h + b1_ref[...], 0.0)
    o_ref[...] = (
        jnp.dot(h, w2_ref[...], preferred_element_type=jnp.float32) + b2_ref[...]
    )


def _mlp(pooled, areas_p, W1, b1, W2, b2):
    return pl.pallas_call(
        _mlp_body,
        grid=(B // _MLP_BB,),
        in_specs=[
            pl.BlockSpec((_MLP_BB, D), lambda i: (i, 0)),
            pl.BlockSpec((_MLP_BB, LP), lambda i: (i, 0)),
            pl.BlockSpec((D, 32), lambda i: (0, 0)),
            pl.BlockSpec((1, 32), lambda i: (0, 0)),
            pl.BlockSpec((32, 3), lambda i: (0, 0)),
            pl.BlockSpec((1, 3), lambda i: (0, 0)),
        ],
        out_specs=pl.BlockSpec((_MLP_BB, 3), lambda i: (i, 0)),
        out_shape=jax.ShapeDtypeStruct((B, 3), jnp.float32),
    )(pooled, areas_p, W1, b1, W2, b2)


def kernel(x, emb, W1, b1, W2, b2):
    types = x[:, 0, :].astype(jnp.int32)
    areas = x[:, 1, :]
    types_p = jnp.pad(types, ((0, 0), (0, LP - L)), mode="edge")
    areas_p = jnp.pad(areas, ((0, 0), (0, LP - L)))
    pooled = _pool_sc(types_p, areas_p, emb)
    return _mlp(pooled, areas_p, W1, b1.reshape(1, 32), W2, b2.reshape(1, 3))
